# baseline (device time: 78197 ns/iter reference)
import jax
import jax.numpy as jnp
from jax import lax
from jax.experimental import pallas as pl
from jax.experimental.pallas import tpu as pltpu

N_DEV = 4
SQ = 1024
SKV = 1024
HQ_PER = 8
DH = 128
BLK = 64
CHUNK = SQ // N_DEV
SCALE = 0.08838834764831843
NEG = -1e9


HALF = CHUNK // 2

PERM_BLOCKS = (2, 5, 8, 11, 14, 0, 3, 6, 9, 12, 15, 1, 4, 7, 10, 13, 0)
SEG = 384
KVP = 64 * len(PERM_BLOCKS)


def _fused_body(
    x_ref, wq_ref, k_ref, v_ref, wo_ref, out_ref,
    q_scr, ctx_scr, rs_send, rs_recv, ag_send0, ag_send1, ag_recv,
    rs_ssem, rs_rsem, ag_ssem, ag_rsem,
):
    my = lax.axis_index("i")

    barrier_sem = pltpu.get_barrier_semaphore()
    for j in (1, 2, 3):
        pl.semaphore_signal(
            barrier_sem, inc=1,
            device_id=(lax.rem(my + j, N_DEV),),
            device_id_type=pl.DeviceIdType.MESH,
        )
    pl.semaphore_wait(barrier_sem, 3)

    def compute_rows(start, nrows):
        xs = x_ref[pl.ds(start, nrows), :]
        q = jnp.dot(xs, wq_ref[...], preferred_element_type=jnp.float32)
        q_scr[pl.ds(0, nrows), :] = (q * SCALE).astype(jnp.bfloat16)
        nb = nrows // BLK

        def bh(i, carry):
            b = i // HQ_PER
            h = lax.rem(i, HQ_PER)
            qb = start // BLK + b
            case = lax.rem(qb, 3)
            m0 = jnp.where(case == 0, 320, jnp.where(case == 1, 0, 704))
            xst = jnp.maximum(
                jnp.where(
                    case == 1,
                    704 + ((qb - 1) // 3) * BLK,
                    ((qb - 2) // 3) * BLK,
                ),
                0,
            )
            xbias = jnp.where(case == 0, NEG, 0.0).astype(jnp.float32)
            m0 = pl.multiple_of(m0, BLK)
            xst = pl.multiple_of(xst, BLK)
            roff = pl.multiple_of(b * BLK, BLK)
            hoff = pl.multiple_of(h * DH, DH)
            q_b = q_scr[pl.ds(roff, BLK), pl.ds(hoff, DH)]
            sm = lax.dot_general(
                q_b, k_ref[h, pl.ds(m0, SEG), :], (((1,), (1,)), ((), ())),
                preferred_element_type=jnp.float32,
            )
            sx = lax.dot_general(
                q_b, k_ref[h, pl.ds(xst, BLK), :], (((1,), (1,)), ((), ())),
                preferred_element_type=jnp.float32,
            ) + xbias
            wm = jnp.exp(sm)
            wx = jnp.exp(sx)
            ws = (
                jnp.sum(wm, axis=1, keepdims=True)
                + jnp.sum(wx, axis=1, keepdims=True)
            )
            ctx = jnp.dot(
                wm.astype(jnp.bfloat16), v_ref[h, pl.ds(m0, SEG), :],
                preferred_element_type=jnp.float32,
            ) + jnp.dot(
                wx.astype(jnp.bfloat16), v_ref[h, pl.ds(xst, BLK), :],
                preferred_element_type=jnp.float32,
            )
            ctx_scr[pl.ds(roff, BLK), pl.ds(hoff, DH)] = (
                ctx * (1.0 / ws)
            ).astype(jnp.bfloat16)
            return carry

        lax.fori_loop(0, nb * HQ_PER, bh, 0)
        return jnp.dot(
            ctx_scr[pl.ds(0, nrows), :], wo_ref[...],
            preferred_element_type=jnp.float32,
        )

    rs = []
    for j in (1, 2, 3):
        tgt = lax.rem(my + j, N_DEV)
        rs_send[j - 1, :, :] = (
            compute_rows(tgt * CHUNK, CHUNK).astype(jnp.bfloat16)
        )
        r = pltpu.make_async_remote_copy(
            src_ref=rs_send.at[j - 1], dst_ref=rs_recv.at[j - 1],
            send_sem=rs_ssem.at[j - 1], recv_sem=rs_rsem.at[j - 1],
            device_id=(tgt,), device_id_type=pl.DeviceIdType.MESH,
        )
        r.start()
        rs.append(r)

    ag = []
    ag_srcs = (ag_send0, ag_send1)
    for half in (0, 1):
        base = my * CHUNK + half * HALF
        acc = compute_rows(base, HALF)
        if half == 0:
            for j in (1, 2, 3):
                rs[j - 1].wait()
        for j in (1, 2, 3):
            acc = acc + rs_recv[
                j - 1, pl.ds(half * HALF, HALF), :
            ].astype(jnp.float32)
        out_ref[0, pl.ds(base, HALF), :] = acc
        ag_srcs[half][...] = acc.astype(jnp.bfloat16)
        for j in (1, 2, 3):
            tgt = lax.rem(my + j, N_DEV)
            a = pltpu.make_async_remote_copy(
                src_ref=ag_srcs[half],
                dst_ref=ag_recv.at[j - 1, half],
                send_sem=ag_ssem.at[j - 1, half],
                recv_sem=ag_rsem.at[j - 1, half],
                device_id=(tgt,), device_id_type=pl.DeviceIdType.MESH,
            )
            a.start()
            ag.append(a)

    for i, a in enumerate(ag):
        a.wait()
        j = i % 3 + 1
        half = i // 3
        oc = lax.rem(my - j + N_DEV, N_DEV)
        out_ref[0, pl.ds(oc * CHUNK + half * HALF, HALF), :] = (
            ag_recv[j - 1, half, :, :].astype(jnp.float32)
        )


def kernel(x, Wq, K_ext, V_ext, Wo):
    my = lax.axis_index("i")

    x2 = x[0].astype(jnp.bfloat16)
    wq = Wq.astype(jnp.bfloat16)
    wo = Wo.astype(jnp.bfloat16)
    k = lax.dynamic_slice_in_dim(K_ext[0], my * HQ_PER, HQ_PER, axis=1)
    v = lax.dynamic_slice_in_dim(V_ext[0], my * HQ_PER, HQ_PER, axis=1)
    k = jnp.transpose(k, (1, 0, 2)).astype(jnp.bfloat16)
    v = jnp.transpose(v, (1, 0, 2)).astype(jnp.bfloat16)
    k = jnp.concatenate(
        [k[:, b * BLK:(b + 1) * BLK, :] for b in PERM_BLOCKS], axis=1
    )
    v = jnp.concatenate(
        [v[:, b * BLK:(b + 1) * BLK, :] for b in PERM_BLOCKS], axis=1
    )

    return pl.pallas_call(
        _fused_body,
        out_shape=jax.ShapeDtypeStruct((1, SQ, 1024), jnp.float32),
        in_specs=[pl.BlockSpec(memory_space=pltpu.VMEM)] * 5,
        out_specs=pl.BlockSpec(memory_space=pltpu.VMEM),
        scratch_shapes=[
            pltpu.VMEM((CHUNK, HQ_PER * DH), jnp.bfloat16),
            pltpu.VMEM((CHUNK, HQ_PER * DH), jnp.bfloat16),
            pltpu.VMEM((N_DEV - 1, CHUNK, 1024), jnp.bfloat16),
            pltpu.VMEM((N_DEV - 1, CHUNK, 1024), jnp.bfloat16),
            pltpu.VMEM((HALF, 1024), jnp.bfloat16),
            pltpu.VMEM((HALF, 1024), jnp.bfloat16),
            pltpu.VMEM((N_DEV - 1, 2, HALF, 1024), jnp.bfloat16),
            pltpu.SemaphoreType.DMA((N_DEV - 1,)),
            pltpu.SemaphoreType.DMA((N_DEV - 1,)),
            pltpu.SemaphoreType.DMA((N_DEV - 1, 2)),
            pltpu.SemaphoreType.DMA((N_DEV - 1, 2)),
        ],
        compiler_params=pltpu.CompilerParams(collective_id=0),
    )(x2, wq, k, v, wo)


# device time: 59809 ns/iter; 1.3074x vs baseline; 1.3074x over previous
import jax
import jax.numpy as jnp
from jax import lax
from jax.experimental import pallas as pl
from jax.experimental.pallas import tpu as pltpu

N_DEV = 4
SQ = 1024
SKV = 1024
HQ_PER = 8
DH = 128
BLK = 64
CHUNK = SQ // N_DEV
SCALE = 0.08838834764831843
NEG = -1e9


HALF = CHUNK // 2

PERM_BLOCKS = (2, 5, 8, 11, 14, 0, 3, 6, 9, 12, 15, 1, 4, 7, 10, 13, 0)
SEG = 384
KVP = 64 * len(PERM_BLOCKS)


def _fused_body(
    x_ref, wq_ref, k_ref, v_ref, wo_ref, out_ref,
    q_scr, ctx_scr, rs_send, rs_recv, ag_send0, ag_send1, ag_recv,
    rs_ssem, rs_rsem, ag_ssem, ag_rsem,
):
    my = lax.axis_index("i")

    q_scr[...] = (
        jnp.dot(x_ref[...], wq_ref[...], preferred_element_type=jnp.float32)
        * SCALE
    ).astype(jnp.bfloat16)

    barrier_sem = pltpu.get_barrier_semaphore()
    for j in (1, 2, 3):
        pl.semaphore_signal(
            barrier_sem, inc=1,
            device_id=(lax.rem(my + j, N_DEV),),
            device_id_type=pl.DeviceIdType.MESH,
        )
    pl.semaphore_wait(barrier_sem, 3)

    def compute_rows(start, nrows):
        nb = nrows // BLK

        def qblock(b, carry):
            qb = start // BLK + b
            case = lax.rem(qb, 3)
            m0 = jnp.where(case == 0, 320, jnp.where(case == 1, 0, 704))
            xst = jnp.maximum(
                jnp.where(
                    case == 1,
                    704 + ((qb - 1) // 3) * BLK,
                    ((qb - 2) // 3) * BLK,
                ),
                0,
            )
            xbias = jnp.where(case == 0, NEG, 0.0).astype(jnp.float32)
            m0 = pl.multiple_of(m0, BLK)
            xst = pl.multiple_of(xst, BLK)
            roff = pl.multiple_of(b * BLK, BLK)
            qoff = pl.multiple_of(start + b * BLK, BLK)
            for h in range(HQ_PER):
                q_b = q_scr[pl.ds(qoff, BLK), h * DH:(h + 1) * DH]
                sm = lax.dot_general(
                    q_b, k_ref[h, pl.ds(m0, SEG), :],
                    (((1,), (1,)), ((), ())),
                    preferred_element_type=jnp.float32,
                )
                sx = lax.dot_general(
                    q_b, k_ref[h, pl.ds(xst, BLK), :],
                    (((1,), (1,)), ((), ())),
                    preferred_element_type=jnp.float32,
                ) + xbias
                wm = jnp.exp(sm)
                wx = jnp.exp(sx)
                ws = (
                    jnp.sum(wm, axis=1, keepdims=True)
                    + jnp.sum(wx, axis=1, keepdims=True)
                )
                ctx = jnp.dot(
                    wm.astype(jnp.bfloat16), v_ref[h, pl.ds(m0, SEG), :],
                    preferred_element_type=jnp.float32,
                ) + jnp.dot(
                    wx.astype(jnp.bfloat16), v_ref[h, pl.ds(xst, BLK), :],
                    preferred_element_type=jnp.float32,
                )
                ctx_scr[pl.ds(roff, BLK), h * DH:(h + 1) * DH] = (
                    ctx * (1.0 / ws)
                ).astype(jnp.bfloat16)
            return carry

        lax.fori_loop(0, nb, qblock, 0)
        return jnp.dot(
            ctx_scr[pl.ds(0, nrows), :], wo_ref[...],
            preferred_element_type=jnp.float32,
        )

    rs = []
    ag = []
    ag_srcs = (ag_send0, ag_send1)
    for p in (0, 1):
        for j in (1, 2, 3):
            tgt = lax.rem(my + j, N_DEV)
            rs_send[j - 1, pl.ds(p * HALF, HALF), :] = (
                compute_rows(tgt * CHUNK + p * HALF, HALF)
                .astype(jnp.bfloat16)
            )
            r = pltpu.make_async_remote_copy(
                src_ref=rs_send.at[j - 1, pl.ds(p * HALF, HALF)],
                dst_ref=rs_recv.at[j - 1, pl.ds(p * HALF, HALF)],
                send_sem=rs_ssem.at[j - 1, p],
                recv_sem=rs_rsem.at[j - 1, p],
                device_id=(tgt,), device_id_type=pl.DeviceIdType.MESH,
            )
            r.start()
            rs.append(r)

        base = my * CHUNK + p * HALF
        acc = compute_rows(base, HALF)
        for idx in range(3):
            rs[p * 3 + idx].wait()
            acc = acc + rs_recv[
                idx, pl.ds(p * HALF, HALF), :
            ].astype(jnp.float32)
        acc16 = acc.astype(jnp.bfloat16)
        out_ref[0, pl.ds(base, HALF), :] = acc16
        ag_srcs[p][...] = acc16
        for j in (1, 2, 3):
            tgt = lax.rem(my + j, N_DEV)
            a = pltpu.make_async_remote_copy(
                src_ref=ag_srcs[p],
                dst_ref=ag_recv.at[j - 1, p],
                send_sem=ag_ssem.at[j - 1, p],
                recv_sem=ag_rsem.at[j - 1, p],
                device_id=(tgt,), device_id_type=pl.DeviceIdType.MESH,
            )
            a.start()
            ag.append(a)

    for i, a in enumerate(ag):
        a.wait()
        p = i // 3
        j = i % 3 + 1
        oc = lax.rem(my - j + N_DEV, N_DEV)
        out_ref[0, pl.ds(oc * CHUNK + p * HALF, HALF), :] = (
            ag_recv[j - 1, p, :, :]
        )


def kernel(x, Wq, K_ext, V_ext, Wo):
    my = lax.axis_index("i")

    x2 = x[0].astype(jnp.bfloat16)
    wq = Wq.astype(jnp.bfloat16)
    wo = Wo.astype(jnp.bfloat16)
    k = lax.dynamic_slice_in_dim(K_ext[0], my * HQ_PER, HQ_PER, axis=1)
    v = lax.dynamic_slice_in_dim(V_ext[0], my * HQ_PER, HQ_PER, axis=1)
    k = jnp.transpose(k, (1, 0, 2)).astype(jnp.bfloat16)
    v = jnp.transpose(v, (1, 0, 2)).astype(jnp.bfloat16)
    k = jnp.concatenate(
        [k[:, b * BLK:(b + 1) * BLK, :] for b in PERM_BLOCKS], axis=1
    )
    v = jnp.concatenate(
        [v[:, b * BLK:(b + 1) * BLK, :] for b in PERM_BLOCKS], axis=1
    )

    return pl.pallas_call(
        _fused_body,
        out_shape=jax.ShapeDtypeStruct((1, SQ, 1024), jnp.bfloat16),
        in_specs=[pl.BlockSpec(memory_space=pltpu.VMEM)] * 5,
        out_specs=pl.BlockSpec(memory_space=pltpu.VMEM),
        scratch_shapes=[
            pltpu.VMEM((SQ, HQ_PER * DH), jnp.bfloat16),
            pltpu.VMEM((CHUNK, HQ_PER * DH), jnp.bfloat16),
            pltpu.VMEM((N_DEV - 1, CHUNK, 1024), jnp.bfloat16),
            pltpu.VMEM((N_DEV - 1, CHUNK, 1024), jnp.bfloat16),
            pltpu.VMEM((HALF, 1024), jnp.bfloat16),
            pltpu.VMEM((HALF, 1024), jnp.bfloat16),
            pltpu.VMEM((N_DEV - 1, 2, HALF, 1024), jnp.bfloat16),
            pltpu.SemaphoreType.DMA((N_DEV - 1, 2)),
            pltpu.SemaphoreType.DMA((N_DEV - 1, 2)),
            pltpu.SemaphoreType.DMA((N_DEV - 1, 2)),
            pltpu.SemaphoreType.DMA((N_DEV - 1, 2)),
        ],
        compiler_params=pltpu.CompilerParams(collective_id=0),
    )(x2, wq, k, v, wo)
